# baseline (speedup 1.0000x reference)
import jax
import jax.numpy as jnp
from jax.experimental import pallas as pl

_NUM_CLASSES = 3
_NUM_YAW = 2
_BOX_DOF = 7
_PIX = 3200


def _proposal_body(x_ref, w_ref, b_ref, cls_ref, reg_ref):
    x = x_ref[0]
    y = jax.lax.dot_general(
        w_ref[...], x,
        dimension_numbers=(((1,), (0,)), ((), ())),
        preferred_element_type=jnp.float32,
    )
    y = y + b_ref[...]
    ncy = _NUM_CLASSES * _NUM_YAW
    cls_ref[0] = y[:ncy]
    t = y.T
    for g in range(ncy):
        lo = ncy + g * _BOX_DOF
        reg_ref[0, g] = t[:, lo:lo + _BOX_DOF]


def kernel(feature_map, W_cls, b_cls, W_reg, b_reg):
    B, C, H, W = feature_map.shape
    ncy = _NUM_CLASSES * _NUM_YAW
    npix = H * W
    Wr = W_reg.reshape(_NUM_CLASSES, _BOX_DOF, _NUM_YAW, C)
    Wr = Wr.transpose(0, 2, 1, 3).reshape(ncy * _BOX_DOF, C)
    br = b_reg.reshape(_NUM_CLASSES, _BOX_DOF, _NUM_YAW)
    br = br.transpose(0, 2, 1).reshape(ncy * _BOX_DOF)
    Wall = jnp.concatenate([W_cls, Wr], axis=0)
    ball = jnp.concatenate([b_cls, br], axis=0)[:, None]

    xflat = feature_map.reshape(B, C, npix)
    P = _PIX
    grid = (B, npix // P)
    cls_out, reg_out = pl.pallas_call(
        _proposal_body,
        grid=grid,
        in_specs=[
            pl.BlockSpec((1, C, P), lambda b, i: (b, 0, i)),
            pl.BlockSpec((ncy * (1 + _BOX_DOF), C), lambda b, i: (0, 0)),
            pl.BlockSpec((ncy * (1 + _BOX_DOF), 1), lambda b, i: (0, 0)),
        ],
        out_specs=[
            pl.BlockSpec((1, ncy, P), lambda b, i: (b, 0, i)),
            pl.BlockSpec((1, ncy, P, _BOX_DOF), lambda b, i: (b, 0, i, 0)),
        ],
        out_shape=[
            jax.ShapeDtypeStruct((B, ncy, npix), jnp.float32),
            jax.ShapeDtypeStruct((B, ncy, npix, _BOX_DOF), jnp.float32),
        ],
    )(xflat, Wall, ball)
    cls_map = cls_out.reshape(B, _NUM_CLASSES, _NUM_YAW, H, W)
    reg_map = reg_out.reshape(B, _NUM_CLASSES, _NUM_YAW, H, W, _BOX_DOF)
    return cls_map, reg_map


# speedup vs baseline: 8.0070x; 8.0070x over previous
import jax
import jax.numpy as jnp
from jax.experimental import pallas as pl

_NUM_CLASSES = 3
_NUM_YAW = 2
_BOX_DOF = 7
_WCHUNK = 16


def _proposal_body(x_ref, w_ref, b_ref, cls_ref, reg_ref):
    H, Wc, C = x_ref.shape[1], x_ref.shape[2], x_ref.shape[3]
    x = x_ref[0].reshape(H * Wc, C)
    y = jax.lax.dot_general(
        x, w_ref[...],
        dimension_numbers=(((1,), (0,)), ((), ())),
        preferred_element_type=jnp.float32,
    )
    z = y.reshape(H, Wc, y.shape[1]).transpose(2, 1, 0)
    z = z + b_ref[...][:, :, None]
    ncy = _NUM_CLASSES * _NUM_YAW
    cls_ref[0] = z[:ncy]
    reg_ref[0] = z[ncy:]


def kernel(feature_map, W_cls, b_cls, W_reg, b_reg):
    B, C, H, W = feature_map.shape
    ncy = _NUM_CLASSES * _NUM_YAW
    nreg = ncy * _BOX_DOF
    Wr = W_reg.reshape(_NUM_CLASSES, _BOX_DOF, _NUM_YAW, C)
    Wr = Wr.transpose(0, 2, 1, 3).reshape(nreg, C)
    br = b_reg.reshape(_NUM_CLASSES, _BOX_DOF, _NUM_YAW)
    br = br.transpose(0, 2, 1).reshape(nreg)
    Wt = jnp.concatenate([W_cls, Wr], axis=0).T
    ball = jnp.concatenate([b_cls, br], axis=0)[:, None]

    xhwc = jnp.transpose(feature_map, (0, 2, 3, 1))
    Wc = _WCHUNK
    grid = (B, W // Wc)
    cls_out, reg_out = pl.pallas_call(
        _proposal_body,
        grid=grid,
        in_specs=[
            pl.BlockSpec((1, H, Wc, C), lambda b, j: (b, 0, j, 0)),
            pl.BlockSpec((C, ncy + nreg), lambda b, j: (0, 0)),
            pl.BlockSpec((ncy + nreg, 1), lambda b, j: (0, 0)),
        ],
        out_specs=[
            pl.BlockSpec((1, ncy, Wc, H), lambda b, j: (b, 0, j, 0)),
            pl.BlockSpec((1, nreg, Wc, H), lambda b, j: (b, 0, j, 0)),
        ],
        out_shape=[
            jax.ShapeDtypeStruct((B, ncy, W, H), jnp.float32),
            jax.ShapeDtypeStruct((B, nreg, W, H), jnp.float32),
        ],
    )(xhwc, Wt, ball)
    cls_map = cls_out.transpose(0, 1, 3, 2).reshape(
        B, _NUM_CLASSES, _NUM_YAW, H, W)
    reg_map = reg_out.reshape(
        B, _NUM_CLASSES, _NUM_YAW, _BOX_DOF, W, H).transpose(0, 1, 2, 5, 4, 3)
    return cls_map, reg_map
